# Initial kernel scaffold; baseline (speedup 1.0000x reference)
import jax
import jax.numpy as jnp
from jax.experimental import pallas as pl


def kernel(input_text, encodings_weight):
    raise NotImplementedError("write your pallas kernel here")


# speedup vs baseline: 6.9927x; 6.9927x over previous
import jax
import jax.numpy as jnp
from jax import lax
from jax.experimental import pallas as pl
from jax.experimental.pallas import tpu as pltpu
from jax.experimental.pallas import tpu_sc as plsc

_NUM_CORES = 2
_NUM_SUBCORES = 16
_NW = _NUM_CORES * _NUM_SUBCORES
_CHUNK = 128


def _gather_body(table_hbm, idx_hbm, out_hbm, idx_v, rows_v, sem):
    wid = lax.axis_index("s") * _NUM_CORES + lax.axis_index("c")
    nchunk = idx_v.shape[0]
    pltpu.sync_copy(idx_hbm.at[wid], idx_v)

    def body(g, carry):
        pltpu.async_copy(table_hbm.at[idx_v.at[g]], rows_v, sem).wait()
        pltpu.sync_copy(rows_v, out_hbm.at[wid, g])
        return carry

    lax.fori_loop(0, nchunk, body, 0)


def kernel(input_text, encodings_weight):
    batch, seq = input_text.shape
    emb = encodings_weight.shape[1]
    n = batch * seq
    per_w = n // _NW
    nchunk = per_w // _CHUNK
    assert per_w * _NW == n and nchunk * _CHUNK == per_w

    idx = input_text.reshape(_NW, nchunk, _CHUNK).astype(jnp.int32)
    mesh = plsc.VectorSubcoreMesh(core_axis_name="c", subcore_axis_name="s")
    out = pl.kernel(
        _gather_body,
        out_type=jax.ShapeDtypeStruct((_NW, nchunk, _CHUNK, emb), jnp.float32),
        mesh=mesh,
        scratch_types=[
            pltpu.VMEM((nchunk, _CHUNK), jnp.int32),
            pltpu.VMEM((_CHUNK, emb), jnp.float32),
            pltpu.SemaphoreType.DMA,
        ],
    )(encodings_weight, idx)
    return out.reshape(batch, seq, emb)


# speedup vs baseline: 9.8324x; 1.4061x over previous
import jax
import jax.numpy as jnp
from jax import lax
from jax.experimental import pallas as pl
from jax.experimental.pallas import tpu as pltpu
from jax.experimental.pallas import tpu_sc as plsc

_NUM_CORES = 2
_NUM_SUBCORES = 16
_NW = _NUM_CORES * _NUM_SUBCORES
_CHUNK = 128


def _gather_body(table_hbm, idx_hbm, out_hbm, idx_v, rows_v, sg, so):
    wid = lax.axis_index("s") * _NUM_CORES + lax.axis_index("c")
    nchunk = idx_v.shape[0]
    pltpu.sync_copy(idx_hbm.at[wid], idx_v)

    def start_gather(g, b):
        pltpu.make_async_copy(
            table_hbm.at[idx_v.at[g]], rows_v.at[b], sg.at[b]
        ).start()

    def wait_gather(g, b):
        pltpu.make_async_copy(
            table_hbm.at[idx_v.at[g]], rows_v.at[b], sg.at[b]
        ).wait()

    def start_out(g, b):
        pltpu.make_async_copy(rows_v.at[b], out_hbm.at[wid, g], so.at[b]).start()

    def wait_out(b):
        pltpu.make_async_copy(rows_v.at[b], out_hbm.at[wid, 0], so.at[b]).wait()

    start_gather(0, 0)

    def body(g0, carry):
        @pl.when(g0 >= 2)
        def _():
            wait_out(1)

        start_gather(g0 + 1, 1)
        wait_gather(g0, 0)
        start_out(g0, 0)

        @pl.when(g0 < nchunk - 2)
        def _():
            wait_out(0)
            start_gather(g0 + 2, 0)

        wait_gather(g0 + 1, 1)
        start_out(g0 + 1, 1)
        return carry

    lax.fori_loop(0, nchunk // 2, lambda i, c: body(i * 2, c), 0)
    wait_out(0)
    wait_out(1)


def kernel(input_text, encodings_weight):
    batch, seq = input_text.shape
    emb = encodings_weight.shape[1]
    n = batch * seq
    per_w = n // _NW
    nchunk = per_w // _CHUNK
    assert per_w * _NW == n and nchunk * _CHUNK == per_w

    idx = input_text.reshape(_NW, nchunk, _CHUNK).astype(jnp.int32)
    mesh = plsc.VectorSubcoreMesh(core_axis_name="c", subcore_axis_name="s")
    out = pl.kernel(
        _gather_body,
        out_type=jax.ShapeDtypeStruct((_NW, nchunk, _CHUNK, emb), jnp.float32),
        mesh=mesh,
        scratch_types=[
            pltpu.VMEM((nchunk, _CHUNK), jnp.int32),
            pltpu.VMEM((2, _CHUNK, emb), jnp.float32),
            pltpu.SemaphoreType.DMA((2,)),
            pltpu.SemaphoreType.DMA((2,)),
        ],
    )(encodings_weight, idx)
    return out.reshape(batch, seq, emb)


# speedup vs baseline: 10.1211x; 1.0294x over previous
import jax
import jax.numpy as jnp
from jax import lax
from jax.experimental import pallas as pl
from jax.experimental.pallas import tpu as pltpu
from jax.experimental.pallas import tpu_sc as plsc

_NUM_CORES = 2
_NUM_SUBCORES = 16
_NW = _NUM_CORES * _NUM_SUBCORES
_CHUNK = 128
_NBUF = 4


def _gather_body(table_hbm, idx_hbm, out_hbm, idx_v, rows_v, sg, so):
    wid = lax.axis_index("s") * _NUM_CORES + lax.axis_index("c")
    nchunk = idx_v.shape[0]
    pltpu.sync_copy(idx_hbm.at[wid], idx_v)

    def start_gather(g, b):
        pltpu.make_async_copy(
            table_hbm.at[idx_v.at[g]], rows_v.at[b], sg.at[b]
        ).start()

    def wait_gather(g, b):
        pltpu.make_async_copy(
            table_hbm.at[idx_v.at[g]], rows_v.at[b], sg.at[b]
        ).wait()

    def start_out(g, b):
        pltpu.make_async_copy(rows_v.at[b], out_hbm.at[wid, g], so.at[b]).start()

    def wait_out(b):
        pltpu.make_async_copy(rows_v.at[b], out_hbm.at[wid, 0], so.at[b]).wait()

    for g in range(_NBUF - 1):
        start_gather(g, g)

    def body(g0, carry):
        for db in range(_NBUF):
            g = g0 + db
            nb = (db + _NBUF - 1) % _NBUF

            @pl.when(g + _NBUF - 1 < nchunk)
            def _():
                @pl.when(g >= 1)
                def _():
                    wait_out(nb)

                start_gather(g + _NBUF - 1, nb)

            wait_gather(g, db)
            start_out(g, db)
        return carry

    lax.fori_loop(0, nchunk // _NBUF, lambda i, c: body(i * _NBUF, c), 0)
    for b in range(_NBUF):
        wait_out(b)


def kernel(input_text, encodings_weight):
    batch, seq = input_text.shape
    emb = encodings_weight.shape[1]
    n = batch * seq
    per_w = n // _NW
    nchunk = per_w // _CHUNK
    assert per_w * _NW == n and nchunk * _CHUNK == per_w

    idx = input_text.reshape(_NW, nchunk, _CHUNK).astype(jnp.int32)
    mesh = plsc.VectorSubcoreMesh(core_axis_name="c", subcore_axis_name="s")
    out = pl.kernel(
        _gather_body,
        out_type=jax.ShapeDtypeStruct((_NW, nchunk, _CHUNK, emb), jnp.float32),
        mesh=mesh,
        scratch_types=[
            pltpu.VMEM((nchunk, _CHUNK), jnp.int32),
            pltpu.VMEM((_NBUF, _CHUNK, emb), jnp.float32),
            pltpu.SemaphoreType.DMA((_NBUF,)),
            pltpu.SemaphoreType.DMA((_NBUF,)),
        ],
    )(encodings_weight, idx)
    return out.reshape(batch, seq, emb)


# speedup vs baseline: 17.3376x; 1.7130x over previous
import jax
import jax.numpy as jnp
from jax import lax
from jax.experimental import pallas as pl
from jax.experimental.pallas import tpu as pltpu
from jax.experimental.pallas import tpu_sc as plsc

_NUM_CORES = 2
_NUM_SUBCORES = 16
_NW = _NUM_CORES * _NUM_SUBCORES
_CHUNK = 128
_NBUF = 2


def _gather_body(table_hbm, idx_hbm, out_hbm, idx_v, rows_v, tab_sh, sg, so):
    sid = lax.axis_index("s")
    wid = sid * _NUM_CORES + lax.axis_index("c")
    nchunk = idx_v.shape[0]
    pltpu.sync_copy(idx_hbm.at[wid], idx_v)
    tab_rows = table_hbm.shape[0] // _NUM_SUBCORES
    pltpu.sync_copy(
        table_hbm.at[pl.ds(sid * tab_rows, tab_rows)],
        tab_sh.at[pl.ds(sid * tab_rows, tab_rows)],
    )
    plsc.subcore_barrier()

    def start_gather(g, b):
        pltpu.make_async_copy(
            tab_sh.at[idx_v.at[g]], rows_v.at[b], sg.at[b]
        ).start()

    def wait_gather(g, b):
        pltpu.make_async_copy(
            tab_sh.at[idx_v.at[g]], rows_v.at[b], sg.at[b]
        ).wait()

    def start_out(g, b):
        pltpu.make_async_copy(rows_v.at[b], out_hbm.at[wid, g], so.at[b]).start()

    def wait_out(b):
        pltpu.make_async_copy(rows_v.at[b], out_hbm.at[wid, 0], so.at[b]).wait()

    for g in range(_NBUF - 1):
        start_gather(g, g)

    def body(g0, carry):
        for db in range(_NBUF):
            g = g0 + db
            nb = (db + _NBUF - 1) % _NBUF

            @pl.when(g + _NBUF - 1 < nchunk)
            def _():
                @pl.when(g >= 1)
                def _():
                    wait_out(nb)

                start_gather(g + _NBUF - 1, nb)

            wait_gather(g, db)
            start_out(g, db)
        return carry

    lax.fori_loop(0, nchunk // _NBUF, lambda i, c: body(i * _NBUF, c), 0)
    for b in range(_NBUF):
        wait_out(b)


def kernel(input_text, encodings_weight):
    batch, seq = input_text.shape
    emb = encodings_weight.shape[1]
    n = batch * seq
    per_w = n // _NW
    nchunk = per_w // _CHUNK
    assert per_w * _NW == n and nchunk * _CHUNK == per_w

    idx = input_text.reshape(_NW, nchunk, _CHUNK).astype(jnp.int32)
    mesh = plsc.VectorSubcoreMesh(core_axis_name="c", subcore_axis_name="s")
    out = pl.kernel(
        _gather_body,
        out_type=jax.ShapeDtypeStruct((_NW, nchunk, _CHUNK, emb), jnp.float32),
        mesh=mesh,
        scratch_types=[
            pltpu.VMEM((nchunk, _CHUNK), jnp.int32),
            pltpu.VMEM((_NBUF, _CHUNK, emb), jnp.float32),
            pltpu.VMEM_SHARED((encodings_weight.shape[0], emb), jnp.float32),
            pltpu.SemaphoreType.DMA((_NBUF,)),
            pltpu.SemaphoreType.DMA((_NBUF,)),
        ],
    )(encodings_weight, idx)
    return out.reshape(batch, seq, emb)
